# baseline (speedup 1.0000x reference)
import jax
import jax.numpy as jnp
from jax.experimental import pallas as pl
from jax.experimental.pallas import tpu as pltpu

_N = 10000
_A = 128
_B = 128
_BM = 400
_STEPS = _N // _BM
_DEPTH = 3


def _fused_kernel(h_ref, w_ref, bias_ref, a_hbm, out_ref, hw_ref, abuf, sems):
    i = pl.program_id(0)

    @pl.when(i == 0)
    def _():
        for k in range(_DEPTH - 1):
            pltpu.make_async_copy(a_hbm.at[pl.ds(k * _BM, _BM), :],
                                  abuf.at[k], sems.at[k]).start()

        def _hw_chunk(c, carry):
            hw_ref[pl.ds(c * 1000, 1000), :] = jnp.dot(
                h_ref[pl.ds(c * 1000, 1000), :], w_ref[...],
                preferred_element_type=jnp.float32,
                precision=jax.lax.Precision.HIGHEST)
            return carry

        jax.lax.fori_loop(0, _N // 1000, _hw_chunk, 0)

    j = i + _DEPTH - 1

    @pl.when(j < _STEPS)
    def _():
        pltpu.make_async_copy(a_hbm.at[pl.ds(j * _BM, _BM), :],
                              abuf.at[j % _DEPTH], sems.at[j % _DEPTH]).start()

    pltpu.make_async_copy(a_hbm.at[pl.ds(i * _BM, _BM), :],
                          abuf.at[i % _DEPTH], sems.at[i % _DEPTH]).wait()
    acc = jnp.dot(abuf[i % _DEPTH], hw_ref[...],
                  preferred_element_type=jnp.float32,
                  precision=jax.lax.Precision.DEFAULT)
    out_ref[...] = acc + bias_ref[...]


def kernel(structure, H, W, bias):
    return pl.pallas_call(
        _fused_kernel,
        out_shape=jax.ShapeDtypeStruct((_N, _B), jnp.float32),
        grid=(_STEPS,),
        in_specs=[
            pl.BlockSpec((_N, _A), lambda i: (0, 0)),
            pl.BlockSpec((_A, _B), lambda i: (0, 0)),
            pl.BlockSpec((1, _B), lambda i: (0, 0)),
            pl.BlockSpec(memory_space=pltpu.MemorySpace.HBM),
        ],
        out_specs=pl.BlockSpec((_BM, _B), lambda i: (i, 0)),
        scratch_shapes=[
            pltpu.VMEM((_N, _B), jnp.float32),
            pltpu.VMEM((_DEPTH, _BM, _N), jnp.float32),
            pltpu.SemaphoreType.DMA((_DEPTH,)),
        ],
        compiler_params=pltpu.CompilerParams(
            dimension_semantics=("arbitrary",),
            vmem_limit_bytes=67108864,
        ),
    )(H, W, bias.reshape(1, _B), structure)


# speedup vs baseline: 1.0309x; 1.0309x over previous
import jax
import jax.numpy as jnp
from jax.experimental import pallas as pl
from jax.experimental.pallas import tpu as pltpu

_N = 10000
_A = 128
_B = 128
_BM = 200
_STEPS = _N // _BM
_DEPTH = 4


def _fused_kernel(h_ref, w_ref, bias_ref, a_hbm, out_ref, hw_ref, abuf, sems):
    i = pl.program_id(0)

    @pl.when(i == 0)
    def _():
        for k in range(_DEPTH - 1):
            pltpu.make_async_copy(a_hbm.at[pl.ds(k * _BM, _BM), :],
                                  abuf.at[k], sems.at[k]).start()
        hw_ref[...] = jnp.dot(h_ref[...], w_ref[...],
                              preferred_element_type=jnp.float32,
                              precision=jax.lax.Precision.HIGHEST
                              ).astype(jnp.bfloat16)

    j = i + _DEPTH - 1

    @pl.when(j < _STEPS)
    def _():
        pltpu.make_async_copy(a_hbm.at[pl.ds(j * _BM, _BM), :],
                              abuf.at[j % _DEPTH], sems.at[j % _DEPTH]).start()

    pltpu.make_async_copy(a_hbm.at[pl.ds(i * _BM, _BM), :],
                          abuf.at[i % _DEPTH], sems.at[i % _DEPTH]).wait()
    acc = jnp.dot(abuf[i % _DEPTH], hw_ref[...],
                  preferred_element_type=jnp.float32,
                  precision=jax.lax.Precision.DEFAULT)
    out_ref[...] = acc + bias_ref[...]


def kernel(structure, H, W, bias):
    return pl.pallas_call(
        _fused_kernel,
        out_shape=jax.ShapeDtypeStruct((_N, _B), jnp.float32),
        grid=(_STEPS,),
        in_specs=[
            pl.BlockSpec((_N, _A), lambda i: (0, 0)),
            pl.BlockSpec((_A, _B), lambda i: (0, 0)),
            pl.BlockSpec((1, _B), lambda i: (0, 0)),
            pl.BlockSpec(memory_space=pltpu.MemorySpace.HBM),
        ],
        out_specs=pl.BlockSpec((_BM, _B), lambda i: (i, 0)),
        scratch_shapes=[
            pltpu.VMEM((_N, _B), jnp.bfloat16),
            pltpu.VMEM((_DEPTH, _BM, _N), jnp.float32),
            pltpu.SemaphoreType.DMA((_DEPTH,)),
        ],
        compiler_params=pltpu.CompilerParams(
            dimension_semantics=("arbitrary",),
            vmem_limit_bytes=67108864,
        ),
    )(H, W, bias.reshape(1, _B), structure)
